# baseline (speedup 1.0000x reference)
import functools

import numpy as np
import jax
import jax.numpy as jnp
from jax import lax
from jax.experimental import pallas as pl
from jax.experimental.pallas import tpu as pltpu
from jax.experimental.pallas import tpu_sc as plsc

_SMOOTH = 0.001
_KAPPA = float(np.sqrt(3.0) / 8.0)

_NW = 32
_RPW = 8
_NR = 10
_W = 288
_NT = 17


def _abs_diff(a, b):
    return jnp.abs(a - b)


def _stencil_pass(buf, o0, emit):
    top0 = [buf[z, 0, pl.ds(o0, 16)] for z in range(4)]
    top1 = [buf[z, 0, pl.ds(o0 + 1, 16)] for z in range(4)]
    xt = [_abs_diff(top0[z], top1[z]) for z in range(4)]
    zt0 = [_abs_diff(top0[p], top0[p + 1]) for p in range(3)]
    zt1 = [_abs_diff(top1[p], top1[p + 1]) for p in range(3)]
    for rr in range(9):
        bot0 = [buf[z, rr + 1, pl.ds(o0, 16)] for z in range(4)]
        bot1 = [buf[z, rr + 1, pl.ds(o0 + 1, 16)] for z in range(4)]
        xb = [_abs_diff(bot0[z], bot1[z]) for z in range(4)]
        zb0 = [_abs_diff(bot0[p], bot0[p + 1]) for p in range(3)]
        zb1 = [_abs_diff(bot1[p], bot1[p + 1]) for p in range(3)]
        xy = [
            xt[z] + xb[z]
            + _abs_diff(top0[z], bot0[z]) + _abs_diff(top1[z], bot1[z])
            for z in range(4)
        ]
        areas = [
            xy[p] + xy[p + 1] + (zt0[p] + zt1[p] + zb0[p] + zb1[p])
            for p in range(3)
        ]
        emit(rr, areas)
        top0, top1, xt, zt0, zt1 = bot0, bot1, xb, zb0, zb1


def _sc_body(pred_hbm, lab_hbm, out_hbm, praw, lraw, sig, lab, pa_strip, accv):
    wid = lax.axis_index("s") * 2 + lax.axis_index("c")
    row0 = wid * _RPW
    st = jnp.maximum(0, jnp.minimum(row0 - 1, 255 - _NR + 1))
    d = row0 - 1 - st

    for z in range(4):
        pltpu.sync_copy(pred_hbm.at[z, pl.ds(st, _NR)], praw.at[z])
        pltpu.sync_copy(lab_hbm.at[z, pl.ds(st, _NR)], lraw.at[z])

    zeros = jnp.zeros((16,), jnp.float32)

    def _stage(u, carry):
        z = u // _NR
        m = u % _NR
        orig = row0 + m - 1
        s = jnp.where(
            jnp.logical_and(orig >= 0, orig <= 255),
            jnp.float32(1.0), jnp.float32(0.0),
        )
        pr = jnp.maximum(0, jnp.minimum(m + d, _NR - 1))
        for ref in (sig, lab):
            ref[z, m, pl.ds(0, 16)] = zeros
            ref[z, m, pl.ds(256, 16)] = zeros
            ref[z, m, pl.ds(272, 16)] = zeros
        for k in range(16):
            v = praw[z, pr, pl.ds(16 * k, 16)]
            sig[z, m, pl.ds(16 * k + 1, 16)] = s / (1.0 + jnp.exp(-v))
            w = lraw[z, pr, pl.ds(16 * k, 16)].astype(jnp.float32)
            lab[z, m, pl.ds(16 * k + 1, 16)] = s * w
        return carry

    lax.fori_loop(0, 4 * _NR, _stage, 0)

    def _pred_pass(t, carry):
        o0 = 16 * t

        def emit(rr, areas):
            for p in range(3):
                pa_strip[9 * p + rr, pl.ds(o0, 16)] = areas[p]

        _stencil_pass(sig, o0, emit)
        return carry

    lax.fori_loop(0, _NT, _pred_pass, 0)

    s8 = jnp.where(wid == _NW - 1, jnp.float32(1.0), jnp.float32(0.0))

    def _label_pass(t, carry):
        acc_n, acc_d = carry
        o0 = 16 * t
        box = [acc_n, acc_d]

        def emit(rr, areas):
            for p in range(3):
                la = areas[p]
                pa = pa_strip[9 * p + rr, pl.ds(o0, 16)]
                tot = pa + la
                both = jnp.logical_and(pa > 0.0, la > 0.0)
                sel = jnp.where(both, tot, 0.0)
                if rr == _RPW:
                    tot = tot * s8
                    sel = sel * s8
                box[0] = box[0] + sel
                box[1] = box[1] + tot

        _stencil_pass(lab, o0, emit)
        return box[0], box[1]

    acc_n, acc_d = lax.fori_loop(0, _NT, _label_pass, (zeros, zeros))
    accv[0] = acc_n * _KAPPA
    accv[1] = acc_d * _KAPPA
    pltpu.sync_copy(accv, out_hbm.at[wid])


_sc_dice = functools.partial(
    pl.kernel,
    out_type=jax.ShapeDtypeStruct((_NW, 2, 16), jnp.float32),
    mesh=plsc.VectorSubcoreMesh(core_axis_name="c", subcore_axis_name="s"),
    scratch_types=[
        pltpu.VMEM((4, _NR, 256), jnp.float32),
        pltpu.VMEM((4, _NR, 256), jnp.int32),
        pltpu.VMEM((4, _NR, _W), jnp.float32),
        pltpu.VMEM((4, _NR, _W), jnp.float32),
        pltpu.VMEM((27, _W), jnp.float32),
        pltpu.VMEM((2, 16), jnp.float32),
    ],
    compiler_params=pltpu.CompilerParams(use_tc_tiling_on_sc=False),
)(_sc_body)


def _combine_body(parts_ref, out_ref):
    parts = parts_ref[...]
    n = jnp.sum(parts[:, 0, :])
    d = jnp.sum(parts[:, 1, :])
    dice = 1.0 - (n + _SMOOTH) / (d + _SMOOTH)
    out_ref[...] = jnp.full((1, 1), dice, jnp.float32)


def kernel(pred, labels):
    B = pred.shape[0]
    dices = []
    for b in range(B):
        parts = _sc_dice(pred[b], labels[b])
        out = pl.pallas_call(
            _combine_body,
            out_shape=jax.ShapeDtypeStruct((1, 1), jnp.float32),
        )(parts)
        dices.append(out[0, 0])
    return jnp.mean(jnp.stack(dices))


# speedup vs baseline: 1.3660x; 1.3660x over previous
import functools

import numpy as np
import jax
import jax.numpy as jnp
from jax import lax
from jax.experimental import pallas as pl
from jax.experimental.pallas import tpu as pltpu
from jax.experimental.pallas import tpu_sc as plsc

_SMOOTH = 0.001
_KAPPA = float(np.sqrt(3.0) / 8.0)

_NW = 32
_RPW = 8
_NR = 10
_W = 288
_NT = 17


def _ad(a, b):
    return jnp.abs(a - b)


def _areas(buf, rr, o_l, o_r):
    tl = [buf[z, rr, pl.ds(o_l, 16)] for z in range(4)]
    tr = [buf[z, rr, pl.ds(o_r, 16)] for z in range(4)]
    bl = [buf[z, rr + 1, pl.ds(o_l, 16)] for z in range(4)]
    br = [buf[z, rr + 1, pl.ds(o_r, 16)] for z in range(4)]
    xy = [
        _ad(tl[z], tr[z]) + _ad(bl[z], br[z])
        + _ad(tl[z], bl[z]) + _ad(tr[z], br[z])
        for z in range(4)
    ]
    return [
        xy[p] + xy[p + 1]
        + (_ad(tl[p], tl[p + 1]) + _ad(tr[p], tr[p + 1])
           + _ad(bl[p], bl[p + 1]) + _ad(br[p], br[p + 1]))
        for p in range(3)
    ]


def _sc_body(pred_hbm, lab_hbm, out_hbm, praw, lraw, sig, lab, accv):
    wid = lax.axis_index("s") * 2 + lax.axis_index("c")
    row0 = wid * _RPW

    @pl.when(wid == 0)
    def _():
        for z in range(4):
            pltpu.sync_copy(pred_hbm.at[z, pl.ds(0, _NR - 1)],
                            praw.at[z, pl.ds(1, _NR - 1)])
            pltpu.sync_copy(lab_hbm.at[z, pl.ds(0, _NR - 1)],
                            lraw.at[z, pl.ds(1, _NR - 1)])

    @pl.when(wid == _NW - 1)
    def _():
        for z in range(4):
            pltpu.sync_copy(pred_hbm.at[z, pl.ds(256 - _NR + 1, _NR - 1)],
                            praw.at[z, pl.ds(0, _NR - 1)])
            pltpu.sync_copy(lab_hbm.at[z, pl.ds(256 - _NR + 1, _NR - 1)],
                            lraw.at[z, pl.ds(0, _NR - 1)])

    @pl.when(jnp.logical_and(wid > 0, wid < _NW - 1))
    def _():
        for z in range(4):
            pltpu.sync_copy(pred_hbm.at[z, pl.ds(row0 - 1, _NR)], praw.at[z])
            pltpu.sync_copy(lab_hbm.at[z, pl.ds(row0 - 1, _NR)], lraw.at[z])

    zeros = jnp.zeros((16,), jnp.float32)

    def _stage(u, carry):
        z = u // _NR
        m = u % _NR
        orig = row0 + m - 1
        valid = jnp.logical_and(orig >= 0, orig <= 255)
        for ref in (sig, lab):
            ref[z, m, pl.ds(0, 16)] = zeros
            ref[z, m, pl.ds(272, 16)] = zeros
        for k in range(16):
            v = praw[z, m, pl.ds(16 * k, 16)]
            sig[z, m, pl.ds(16 * k + 16, 16)] = jnp.where(
                valid, 1.0 / (1.0 + jnp.exp(-v)), zeros)
            w = lraw[z, m, pl.ds(16 * k, 16)].astype(jnp.float32)
            lab[z, m, pl.ds(16 * k + 16, 16)] = jnp.where(valid, w, zeros)
        return carry

    lax.fori_loop(0, 4 * _NR, _stage, 0)

    s8 = jnp.where(wid == _NW - 1, jnp.float32(1.0), jnp.float32(0.0))

    def _row_step(rr, carry):
        acc_n, acc_d = carry
        s = jnp.where(rr < _RPW, jnp.float32(1.0), s8)
        for t in range(_NT):
            o_l = 16 * t + 15
            o_r = 16 * t + 16
            pa3 = _areas(sig, rr, o_l, o_r)
            la3 = _areas(lab, rr, o_l, o_r)
            for p in range(3):
                tot = pa3[p] + la3[p]
                both = jnp.logical_and(pa3[p] > 0.0, la3[p] > 0.0)
                acc_n = acc_n + s * jnp.where(both, tot, 0.0)
                acc_d = acc_d + s * tot
        return acc_n, acc_d

    acc_n, acc_d = lax.fori_loop(0, 9, _row_step, (zeros, zeros))
    accv[0] = acc_n * _KAPPA
    accv[1] = acc_d * _KAPPA
    pltpu.sync_copy(accv, out_hbm.at[wid])


_sc_dice = functools.partial(
    pl.kernel,
    out_type=jax.ShapeDtypeStruct((_NW, 2, 16), jnp.float32),
    mesh=plsc.VectorSubcoreMesh(core_axis_name="c", subcore_axis_name="s"),
    scratch_types=[
        pltpu.VMEM((4, _NR, 256), jnp.float32),
        pltpu.VMEM((4, _NR, 256), jnp.int32),
        pltpu.VMEM((4, _NR, _W), jnp.float32),
        pltpu.VMEM((4, _NR, _W), jnp.float32),
        pltpu.VMEM((2, 16), jnp.float32),
    ],
    compiler_params=pltpu.CompilerParams(use_tc_tiling_on_sc=False),
)(_sc_body)


def _combine_body(parts_ref, out_ref):
    parts = parts_ref[...]
    n = jnp.sum(parts[:, 0, :])
    d = jnp.sum(parts[:, 1, :])
    dice = 1.0 - (n + _SMOOTH) / (d + _SMOOTH)
    out_ref[...] = jnp.full((1, 1), dice, jnp.float32)


def kernel(pred, labels):
    B = pred.shape[0]
    dices = []
    for b in range(B):
        parts = _sc_dice(pred[b], labels[b])
        out = pl.pallas_call(
            _combine_body,
            out_shape=jax.ShapeDtypeStruct((1, 1), jnp.float32),
        )(parts)
        dices.append(out[0, 0])
    return jnp.mean(jnp.stack(dices))
